# baseline (speedup 1.0000x reference)
import functools

import jax
import jax.numpy as jnp
from jax import lax
from jax.experimental import pallas as pl
from jax.experimental.pallas import tpu as pltpu
from jax.experimental.pallas import tpu_sc as plsc

S, D, F, E, K = 2048, 768, 768, 8, 2
TILE = 256
NT = (S * K + E * (TILE - 1) + TILE - 1) // TILE
P = NT * TILE
NC, NS = 2, 16
NW = NC * NS
C = S // NW
WL = 16
NSLOT = 4
LOOKAHEAD = 3


def _router_body(x_ref, gw_ref, gb_ref, d0_ref, d1_ref, te_ref, wrow_ref):
    x = x_ref[...]
    logits = jnp.dot(x, gw_ref[...], preferred_element_type=jnp.float32)
    logits = logits + gb_ref[...][None, :]

    lane = lax.broadcasted_iota(jnp.int32, (S, E), 1)
    neg_inf = jnp.float32(-jnp.inf)
    big = jnp.int32(E + 1)

    m1 = jnp.max(logits, axis=1, keepdims=True)
    i1 = jnp.min(jnp.where(logits == m1, lane, big), axis=1)
    oh1 = (lane == i1[:, None])
    masked = jnp.where(oh1, neg_inf, logits)
    m2 = jnp.max(masked, axis=1, keepdims=True)
    i2 = jnp.min(jnp.where(masked == m2, lane, big), axis=1)
    oh2 = (lane == i2[:, None])

    r = jnp.exp(m2 - m1)
    wa = 1.0 / (1.0 + r)
    wb = 1.0 - wa

    erow = lax.broadcasted_iota(jnp.int32, (E, S), 0)
    oh1T = erow == i1[None, :]
    oh2T = erow == i2[None, :]
    ohsumT = oh1T.astype(jnp.float32) + oh2T.astype(jnp.float32)
    c = ohsumT
    sh = 1
    while sh < S:
        c = c + jnp.concatenate(
            [jnp.zeros((E, sh), jnp.float32), c[:, :-sh]], axis=1)
        sh *= 2
    excl = c - ohsumT
    rank0 = jnp.sum(jnp.where(oh1T, excl, 0.0), axis=0).astype(jnp.int32)
    rank1 = jnp.sum(jnp.where(oh2T, excl, 0.0), axis=0).astype(jnp.int32)

    totals = c[:, S - 1:S].astype(jnp.int32)
    ps = ((totals + (TILE - 1)) // TILE) * TILE
    pe = ps
    sh = 1
    while sh < E:
        pe = pe + jnp.concatenate(
            [jnp.zeros((sh, 1), jnp.int32), pe[:-sh, :]], axis=0)
        sh *= 2
    offs = pe - ps

    dest0 = jnp.sum(jnp.where(oh1T, offs, 0), axis=0) + rank0
    dest1 = jnp.sum(jnp.where(oh2T, offs, 0), axis=0) + rank1

    pe7 = pe[E - 1:E, :]
    tstart = lax.broadcasted_iota(jnp.int32, (1, NT), 1) * TILE
    tstart = jnp.minimum(tstart, pe7 - 1)
    te = jnp.sum((tstart >= pe).astype(jnp.int32), axis=0)
    te = jnp.minimum(te, E - 1)
    nvalid = (pe7 // TILE).reshape((1,))

    present = ps > 0
    presI = present.astype(jnp.int32)
    oscan = presI
    sh = 1
    while sh < E:
        oscan = oscan + jnp.concatenate(
            [jnp.zeros((sh, 1), jnp.int32), oscan[:-sh, :]], axis=0)
        sh *= 2
    ordE = oscan - presI
    erowT = lax.broadcasted_iota(jnp.int32, (E, NT), 0)
    ohT = erowT == te[None, :]
    ordT = jnp.sum(jnp.where(ohT, ordE, 0), axis=0)
    rlane = lax.broadcasted_iota(jnp.int32, (E, E), 1)
    eidx = lax.broadcasted_iota(jnp.int32, (E, E), 0)
    pmat = (ordE == rlane) & present
    plist = jnp.sum(jnp.where(pmat, eidx + 1, 0), axis=0) - 1

    d0_ref[...] = dest0
    d1_ref[...] = dest1
    te_ref[...] = jnp.concatenate(
        [te, nvalid, jnp.zeros((32 - NT - 1,), jnp.int32), ordT, plist])
    wrow_ref[...] = jnp.concatenate(
        [jnp.broadcast_to(wa, (S, WL)), jnp.broadcast_to(wb, (S, WL))],
        axis=0)


def _router(x2, gate_w, gate_b):
    return pl.pallas_call(
        _router_body,
        out_shape=[
            jax.ShapeDtypeStruct((S,), jnp.int32),
            jax.ShapeDtypeStruct((S,), jnp.int32),
            jax.ShapeDtypeStruct((64,), jnp.int32),
            jax.ShapeDtypeStruct((2 * S, WL), jnp.float32),
        ],
    )(x2, gate_w, gate_b)


def _dispatch_body(x_hbm, d0_hbm, d1_hbm, xs_hbm, i0_v, i1_v, rows_v, sem):
    wid = lax.axis_index("s") * NC + lax.axis_index("c")
    base = wid * C
    pltpu.sync_copy(d0_hbm.at[pl.ds(base, C)], i0_v)
    pltpu.sync_copy(d1_hbm.at[pl.ds(base, C)], i1_v)
    pltpu.sync_copy(x_hbm.at[pl.ds(base, C)], rows_v)
    c0 = pltpu.async_copy(rows_v, xs_hbm.at[i0_v], sem)
    c1 = pltpu.async_copy(rows_v, xs_hbm.at[i1_v], sem)
    c0.wait()
    c1.wait()


def _dispatch(x2, d0, d1):
    mesh = plsc.VectorSubcoreMesh(core_axis_name="c", subcore_axis_name="s")
    return pl.kernel(
        _dispatch_body,
        out_type=jax.ShapeDtypeStruct((P, D), jnp.float32),
        mesh=mesh,
        scratch_types=[
            pltpu.VMEM((C,), jnp.int32),
            pltpu.VMEM((C,), jnp.int32),
            pltpu.VMEM((C, D), jnp.float32),
            pltpu.SemaphoreType.DMA,
        ],
    )(x2, d0, d1)


def _ffn_body(te_ref, xs_ref, W1_ref, b1_ref, W2_ref, b2_ref,
              Wg_ref, bg_ref, Wv_ref, bv_ref, W3_ref, b3_ref, ys_ref,
              w1_s, w2_s, wg_s, wv_s, w3_s, sems):
    i = pl.program_id(0)
    my_te = te_ref[i]
    im1 = jnp.maximum(i - 1, 0)
    isfirst = jnp.logical_or(i == 0, te_ref[im1] != my_te)
    ordv = te_ref[32 + i]

    wlist = ((W1_ref, w1_s), (W2_ref, w2_s), (Wg_ref, wg_s),
             (Wv_ref, wv_s), (W3_ref, w3_s))

    def issue(fe, slot):
        @pl.when(fe >= 0)
        def _():
            for hbm, scr in wlist:
                pltpu.async_copy(hbm.at[pl.ds(fe, 1)],
                                 scr.at[pl.ds(slot, 1)], sems.at[slot])

    @pl.when(i == 0)
    def _():
        issue(te_ref[56], 0)
        issue(te_ref[57], 1)
        issue(te_ref[58], 2)

    @pl.when(isfirst)
    def _():
        rf = ordv + LOOKAHEAD
        fe = jnp.where(rf < E, te_ref[56 + jnp.minimum(rf, E - 1)],
                       jnp.int32(-1))
        issue(fe, lax.rem(rf, NSLOT))
        slot = lax.rem(ordv, NSLOT)
        for hbm, scr in wlist:
            pltpu.make_async_copy(hbm.at[pl.ds(0, 1)],
                                  scr.at[pl.ds(slot, 1)],
                                  sems.at[slot]).wait()

    @pl.when(i < te_ref[NT])
    def _():
        dot = functools.partial(jnp.dot, precision=lax.Precision.DEFAULT,
                                preferred_element_type=jnp.float32)
        sl = lax.rem(ordv, NSLOT)
        e = my_te
        xt = xs_ref[...]
        h = ((dot(xt, w1_s[pl.ds(sl, 1)][0]) + b1_ref[pl.ds(e, 1), :])
             * (dot(xt, w2_s[pl.ds(sl, 1)][0]) + b2_ref[pl.ds(e, 1), :]))
        a = dot(h, wg_s[pl.ds(sl, 1)][0]) + bg_ref[pl.ds(e, 1), :]
        silu = a / (1.0 + jnp.exp(-a))
        g = silu * (dot(h, wv_s[pl.ds(sl, 1)][0]) + bv_ref[pl.ds(e, 1), :])
        eo = dot(g, w3_s[pl.ds(sl, 1)][0]) + b3_ref[pl.ds(e, 1), :]
        ys_ref[...] = eo


def _ffn(te_arr, xs, W1, b1, W2, b2, Wg, bg, Wv, bv, W3, b3):
    def bspec(shape):
        return pl.BlockSpec(shape, lambda i, te: (0, 0))

    def any_spec():
        return pl.BlockSpec(memory_space=pl.ANY)

    grid_spec = pltpu.PrefetchScalarGridSpec(
        num_scalar_prefetch=1,
        grid=(NT,),
        in_specs=[
            pl.BlockSpec((TILE, D), lambda i, te: (i, 0)),
            any_spec(), bspec(b1.shape),
            any_spec(), bspec(b2.shape),
            any_spec(), bspec(bg.shape),
            any_spec(), bspec(bv.shape),
            any_spec(), bspec(b3.shape),
        ],
        out_specs=pl.BlockSpec((TILE, D), lambda i, te: (i, 0)),
        scratch_shapes=[
            pltpu.VMEM((NSLOT, D, F), jnp.float32),
            pltpu.VMEM((NSLOT, D, F), jnp.float32),
            pltpu.VMEM((NSLOT, F, F), jnp.float32),
            pltpu.VMEM((NSLOT, F, F), jnp.float32),
            pltpu.VMEM((NSLOT, F, D), jnp.float32),
            pltpu.SemaphoreType.DMA((NSLOT,)),
        ],
    )
    return pl.pallas_call(
        _ffn_body,
        grid_spec=grid_spec,
        out_shape=jax.ShapeDtypeStruct((P, D), jnp.float32),
    )(te_arr, xs, W1, b1, W2, b2, Wg, bg, Wv, bv, W3, b3)


def _combine_body(ys_hbm, d0_hbm, d1_hbm, wr_hbm, out_hbm,
                  i_v, j_v, a_v, b_v, w0_v, w1_v, sem):
    wid = lax.axis_index("s") * NC + lax.axis_index("c")
    base = wid * C
    pltpu.sync_copy(d0_hbm.at[pl.ds(base, C)], i_v)
    pltpu.sync_copy(d1_hbm.at[pl.ds(base, C)], j_v)
    c0 = pltpu.async_copy(ys_hbm.at[i_v], a_v, sem)
    c1 = pltpu.async_copy(ys_hbm.at[j_v], b_v, sem)
    pltpu.sync_copy(wr_hbm.at[pl.ds(base, C)], w0_v)
    pltpu.sync_copy(wr_hbm.at[pl.ds(S + base, C)], w1_v)
    c0.wait()
    c1.wait()

    def body_j(j, carry):
        wa = w0_v[j, :]
        wb = w1_v[j, :]
        for q in range(D // 16):
            sl = pl.ds(q * 16, 16)
            a_v[j, sl] = a_v[j, sl] * wa + b_v[j, sl] * wb
        return carry

    lax.fori_loop(0, C, body_j, 0)
    pltpu.sync_copy(a_v, out_hbm.at[pl.ds(base, C)])


def _combine(ys, d0, d1, wrow):
    mesh = plsc.VectorSubcoreMesh(core_axis_name="c", subcore_axis_name="s")
    return pl.kernel(
        _combine_body,
        out_type=jax.ShapeDtypeStruct((S, D), jnp.float32),
        mesh=mesh,
        scratch_types=[
            pltpu.VMEM((C,), jnp.int32),
            pltpu.VMEM((C,), jnp.int32),
            pltpu.VMEM((C, D), jnp.float32),
            pltpu.VMEM((C, D), jnp.float32),
            pltpu.VMEM((C, WL), jnp.float32),
            pltpu.VMEM((C, WL), jnp.float32),
            pltpu.SemaphoreType.DMA,
        ],
    )(ys, d0, d1, wrow)


@jax.jit
def kernel(x, gate_w, gate_b, W1, b1, W2, b2, Wg, bg, Wv, bv, W3, b3):
    x2 = x.reshape(S, D)
    d0, d1, te_arr, wrow = _router(x2, gate_w, gate_b)
    xs = _dispatch(x2, d0, d1)
    ys = _ffn(te_arr, xs, W1, b1, W2, b2, Wg, bg, Wv, bv, W3, b3)
    out2 = _combine(ys, d0, d1, wrow)
    return out2.reshape(1, S, D)


# speedup vs baseline: 1.0214x; 1.0214x over previous
import functools

import jax
import jax.numpy as jnp
from jax import lax
from jax.experimental import pallas as pl
from jax.experimental.pallas import tpu as pltpu
from jax.experimental.pallas import tpu_sc as plsc

S, D, F, E, K = 2048, 768, 768, 8, 2
TILE = 256
NT = (S * K + E * (TILE - 1) + TILE - 1) // TILE
P = NT * TILE
NC, NS = 2, 16
NW = NC * NS
C = S // NW
WL = 16
NSLOT = 3
LOOKAHEAD = 2


def _router_body(x_ref, gw_ref, gb_ref, d0_ref, d1_ref, te_ref, wrow_ref):
    x = x_ref[...]
    logits = jnp.dot(x, gw_ref[...], preferred_element_type=jnp.float32)
    logits = logits + gb_ref[...][None, :]

    lane = lax.broadcasted_iota(jnp.int32, (S, E), 1)
    neg_inf = jnp.float32(-jnp.inf)
    big = jnp.int32(E + 1)

    m1 = jnp.max(logits, axis=1, keepdims=True)
    i1 = jnp.min(jnp.where(logits == m1, lane, big), axis=1)
    oh1 = (lane == i1[:, None])
    masked = jnp.where(oh1, neg_inf, logits)
    m2 = jnp.max(masked, axis=1, keepdims=True)
    i2 = jnp.min(jnp.where(masked == m2, lane, big), axis=1)
    oh2 = (lane == i2[:, None])

    r = jnp.exp(m2 - m1)
    wa = 1.0 / (1.0 + r)
    wb = 1.0 - wa

    erow = lax.broadcasted_iota(jnp.int32, (E, S), 0)
    oh1T = erow == i1[None, :]
    oh2T = erow == i2[None, :]
    ohsumT = oh1T.astype(jnp.float32) + oh2T.astype(jnp.float32)
    c = ohsumT
    sh = 1
    while sh < S:
        c = c + jnp.concatenate(
            [jnp.zeros((E, sh), jnp.float32), c[:, :-sh]], axis=1)
        sh *= 2
    excl = c - ohsumT
    rank0 = jnp.sum(jnp.where(oh1T, excl, 0.0), axis=0).astype(jnp.int32)
    rank1 = jnp.sum(jnp.where(oh2T, excl, 0.0), axis=0).astype(jnp.int32)

    totals = c[:, S - 1:S].astype(jnp.int32)
    ps = ((totals + (TILE - 1)) // TILE) * TILE
    pe = ps
    sh = 1
    while sh < E:
        pe = pe + jnp.concatenate(
            [jnp.zeros((sh, 1), jnp.int32), pe[:-sh, :]], axis=0)
        sh *= 2
    offs = pe - ps

    dest0 = jnp.sum(jnp.where(oh1T, offs, 0), axis=0) + rank0
    dest1 = jnp.sum(jnp.where(oh2T, offs, 0), axis=0) + rank1

    pe7 = pe[E - 1:E, :]
    tstart = lax.broadcasted_iota(jnp.int32, (1, NT), 1) * TILE
    tstart = jnp.minimum(tstart, pe7 - 1)
    te = jnp.sum((tstart >= pe).astype(jnp.int32), axis=0)
    te = jnp.minimum(te, E - 1)
    nvalid = (pe7 // TILE).reshape((1,))

    present = ps > 0
    presI = present.astype(jnp.int32)
    oscan = presI
    sh = 1
    while sh < E:
        oscan = oscan + jnp.concatenate(
            [jnp.zeros((sh, 1), jnp.int32), oscan[:-sh, :]], axis=0)
        sh *= 2
    ordE = oscan - presI
    erowT = lax.broadcasted_iota(jnp.int32, (E, NT), 0)
    ohT = erowT == te[None, :]
    ordT = jnp.sum(jnp.where(ohT, ordE, 0), axis=0)
    rlane = lax.broadcasted_iota(jnp.int32, (E, E), 1)
    eidx = lax.broadcasted_iota(jnp.int32, (E, E), 0)
    pmat = (ordE == rlane) & present
    plist = jnp.sum(jnp.where(pmat, eidx + 1, 0), axis=0) - 1

    d0_ref[...] = dest0
    d1_ref[...] = dest1
    te_ref[...] = jnp.concatenate(
        [te, nvalid, jnp.zeros((32 - NT - 1,), jnp.int32), ordT, plist])
    wrow_ref[...] = jnp.concatenate(
        [jnp.broadcast_to(wa, (S, WL)), jnp.broadcast_to(wb, (S, WL))],
        axis=0)


def _router(x2, gate_w, gate_b):
    return pl.pallas_call(
        _router_body,
        out_shape=[
            jax.ShapeDtypeStruct((S,), jnp.int32),
            jax.ShapeDtypeStruct((S,), jnp.int32),
            jax.ShapeDtypeStruct((64,), jnp.int32),
            jax.ShapeDtypeStruct((2 * S, WL), jnp.float32),
        ],
    )(x2, gate_w, gate_b)


def _dispatch_body(x_hbm, d0_hbm, d1_hbm, xs_hbm, i0_v, i1_v, rows_v, sem):
    wid = lax.axis_index("s") * NC + lax.axis_index("c")
    base = wid * C
    pltpu.sync_copy(d0_hbm.at[pl.ds(base, C)], i0_v)
    pltpu.sync_copy(d1_hbm.at[pl.ds(base, C)], i1_v)
    pltpu.sync_copy(x_hbm.at[pl.ds(base, C)], rows_v)
    c0 = pltpu.async_copy(rows_v, xs_hbm.at[i0_v], sem)
    c1 = pltpu.async_copy(rows_v, xs_hbm.at[i1_v], sem)
    c0.wait()
    c1.wait()


def _dispatch(x2, d0, d1):
    mesh = plsc.VectorSubcoreMesh(core_axis_name="c", subcore_axis_name="s")
    return pl.kernel(
        _dispatch_body,
        out_type=jax.ShapeDtypeStruct((P, D), jnp.float32),
        mesh=mesh,
        scratch_types=[
            pltpu.VMEM((C,), jnp.int32),
            pltpu.VMEM((C,), jnp.int32),
            pltpu.VMEM((C, D), jnp.float32),
            pltpu.SemaphoreType.DMA,
        ],
    )(x2, d0, d1)


def _ffn_body(te_ref, xs_ref, W1_ref, b1_ref, W2_ref, b2_ref,
              Wg_ref, bg_ref, Wv_ref, bv_ref, W3_ref, b3_ref, ys_ref,
              w1_s, w2_s, wg_s, wv_s, w3_s, sems):
    i = pl.program_id(0)
    my_te = te_ref[i]
    im1 = jnp.maximum(i - 1, 0)
    isfirst = jnp.logical_or(i == 0, te_ref[im1] != my_te)
    ordv = te_ref[32 + i]

    wlist = ((W1_ref, w1_s), (W2_ref, w2_s), (Wg_ref, wg_s),
             (Wv_ref, wv_s), (W3_ref, w3_s))

    def issue(fe, slot):
        @pl.when(fe >= 0)
        def _():
            for hbm, scr in wlist:
                pltpu.async_copy(hbm.at[pl.ds(fe, 1)],
                                 scr.at[pl.ds(slot, 1)], sems.at[slot])

    @pl.when(i == 0)
    def _():
        issue(te_ref[56], 0)
        issue(te_ref[57], 1)

    @pl.when(isfirst)
    def _():
        rf = ordv + LOOKAHEAD
        fe = jnp.where(rf < E, te_ref[56 + jnp.minimum(rf, E - 1)],
                       jnp.int32(-1))
        issue(fe, lax.rem(rf, NSLOT))
        slot = lax.rem(ordv, NSLOT)
        for hbm, scr in wlist:
            pltpu.make_async_copy(hbm.at[pl.ds(0, 1)],
                                  scr.at[pl.ds(slot, 1)],
                                  sems.at[slot]).wait()

    @pl.when(i < te_ref[NT])
    def _():
        dot = functools.partial(jnp.dot, precision=lax.Precision.DEFAULT,
                                preferred_element_type=jnp.float32)
        sl = lax.rem(ordv, NSLOT)
        e = my_te
        xt = xs_ref[...]
        h = ((dot(xt, w1_s[pl.ds(sl, 1)][0]) + b1_ref[pl.ds(e, 1), :])
             * (dot(xt, w2_s[pl.ds(sl, 1)][0]) + b2_ref[pl.ds(e, 1), :]))
        a = dot(h, wg_s[pl.ds(sl, 1)][0]) + bg_ref[pl.ds(e, 1), :]
        silu = a / (1.0 + jnp.exp(-a))
        g = silu * (dot(h, wv_s[pl.ds(sl, 1)][0]) + bv_ref[pl.ds(e, 1), :])
        eo = dot(g, w3_s[pl.ds(sl, 1)][0]) + b3_ref[pl.ds(e, 1), :]
        ys_ref[...] = eo


def _ffn(te_arr, xs, W1, b1, W2, b2, Wg, bg, Wv, bv, W3, b3):
    def bspec(shape):
        return pl.BlockSpec(shape, lambda i, te: (0, 0))

    def any_spec():
        return pl.BlockSpec(memory_space=pl.ANY)

    grid_spec = pltpu.PrefetchScalarGridSpec(
        num_scalar_prefetch=1,
        grid=(NT,),
        in_specs=[
            pl.BlockSpec((TILE, D), lambda i, te: (i, 0)),
            any_spec(), bspec(b1.shape),
            any_spec(), bspec(b2.shape),
            any_spec(), bspec(bg.shape),
            any_spec(), bspec(bv.shape),
            any_spec(), bspec(b3.shape),
        ],
        out_specs=pl.BlockSpec((TILE, D), lambda i, te: (i, 0)),
        scratch_shapes=[
            pltpu.VMEM((NSLOT, D, F), jnp.float32),
            pltpu.VMEM((NSLOT, D, F), jnp.float32),
            pltpu.VMEM((NSLOT, F, F), jnp.float32),
            pltpu.VMEM((NSLOT, F, F), jnp.float32),
            pltpu.VMEM((NSLOT, F, D), jnp.float32),
            pltpu.SemaphoreType.DMA((NSLOT,)),
        ],
    )
    return pl.pallas_call(
        _ffn_body,
        grid_spec=grid_spec,
        out_shape=jax.ShapeDtypeStruct((P, D), jnp.float32),
    )(te_arr, xs, W1, b1, W2, b2, Wg, bg, Wv, bv, W3, b3)


def _combine_body(ys_hbm, d0_hbm, d1_hbm, wr_hbm, out_hbm,
                  i_v, j_v, a_v, b_v, w0_v, w1_v, sem):
    wid = lax.axis_index("s") * NC + lax.axis_index("c")
    base = wid * C
    pltpu.sync_copy(d0_hbm.at[pl.ds(base, C)], i_v)
    pltpu.sync_copy(d1_hbm.at[pl.ds(base, C)], j_v)
    c0 = pltpu.async_copy(ys_hbm.at[i_v], a_v, sem)
    c1 = pltpu.async_copy(ys_hbm.at[j_v], b_v, sem)
    pltpu.sync_copy(wr_hbm.at[pl.ds(base, C)], w0_v)
    pltpu.sync_copy(wr_hbm.at[pl.ds(S + base, C)], w1_v)
    c0.wait()
    c1.wait()

    def body_j(j, carry):
        wa = w0_v[j, :]
        wb = w1_v[j, :]
        for q in range(D // 16):
            sl = pl.ds(q * 16, 16)
            a_v[j, sl] = a_v[j, sl] * wa + b_v[j, sl] * wb
        return carry

    lax.fori_loop(0, C, body_j, 0)
    pltpu.sync_copy(a_v, out_hbm.at[pl.ds(base, C)])


def _combine(ys, d0, d1, wrow):
    mesh = plsc.VectorSubcoreMesh(core_axis_name="c", subcore_axis_name="s")
    return pl.kernel(
        _combine_body,
        out_type=jax.ShapeDtypeStruct((S, D), jnp.float32),
        mesh=mesh,
        scratch_types=[
            pltpu.VMEM((C,), jnp.int32),
            pltpu.VMEM((C,), jnp.int32),
            pltpu.VMEM((C, D), jnp.float32),
            pltpu.VMEM((C, D), jnp.float32),
            pltpu.VMEM((C, WL), jnp.float32),
            pltpu.VMEM((C, WL), jnp.float32),
            pltpu.SemaphoreType.DMA,
        ],
    )(ys, d0, d1, wrow)


@jax.jit
def kernel(x, gate_w, gate_b, W1, b1, W2, b2, Wg, bg, Wv, bv, W3, b3):
    x2 = x.reshape(S, D)
    d0, d1, te_arr, wrow = _router(x2, gate_w, gate_b)
    xs = _dispatch(x2, d0, d1)
    ys = _ffn(te_arr, xs, W1, b1, W2, b2, Wg, bg, Wv, bv, W3, b3)
    out2 = _combine(ys, d0, d1, wrow)
    return out2.reshape(1, S, D)
